# baseline (speedup 1.0000x reference)
import functools

import jax
import jax.numpy as jnp
import numpy as np
from jax.experimental import pallas as pl
from jax.experimental.pallas import tpu as pltpu
from jax.sharding import Mesh, PartitionSpec as P

if hasattr(jax, "shard_map"):
    _shard_map = functools.partial(jax.shard_map, check_vma=False)
else:
    from jax.experimental.shard_map import shard_map as _esm
    _shard_map = functools.partial(_esm, check_rep=False)


def _moe_body(off_ref, x_ref, wg_ref, up_ref, gate_ref, down_ref, out_ref, w_scr):
    e = pl.program_id(0)

    @pl.when(e == 0)
    def _compute_gating():
        xf = x_ref[...].astype(jnp.float32)
        logits = jnp.dot(xf, wg_ref[...], preferred_element_type=jnp.float32)
        m = jnp.max(logits, axis=-1, keepdims=True)
        p = jnp.exp(logits - m)
        g = p / jnp.sum(p, axis=-1, keepdims=True)
        col = jax.lax.broadcasted_iota(jnp.int32, g.shape, 1)
        i1 = jnp.argmax(g, axis=-1)[:, None]
        oh1 = col == i1
        i2 = jnp.argmax(jnp.where(oh1, -1.0, g), axis=-1)[:, None]
        oh2 = col == i2
        w_scr[...] = jnp.where(oh1 | oh2, g, 0.0)

    xb = x_ref[...]
    up_w = up_ref[0]
    gate_w = gate_ref[0]
    down_w = down_ref[0]
    dn = (((1,), (1,)), ((), ()))
    up = jax.lax.dot_general(xb, up_w, dn, preferred_element_type=jnp.float32)
    gate = jax.lax.dot_general(xb, gate_w, dn, preferred_element_type=jnp.float32)
    hidden = (gate * jax.nn.sigmoid(gate) * up).astype(jnp.bfloat16)
    y = jax.lax.dot_general(hidden, down_w, dn, preferred_element_type=jnp.float32)
    wcol = jax.lax.broadcasted_iota(jnp.int32, w_scr.shape, 1)
    we = jnp.sum(jnp.where(wcol == off_ref[0] + e, w_scr[...], 0.0), axis=1,
                 keepdims=True)
    contrib = we * y

    @pl.when(e == 0)
    def _init():
        out_ref[...] = contrib

    @pl.when(e > 0)
    def _acc():
        out_ref[...] += contrib


def _local_moe(off, x, W_gate, up_proj, gate_proj, down_proj):
    S, D = x.shape
    E_loc, H, _ = up_proj.shape
    E = W_gate.shape[1]
    return pl.pallas_call(
        _moe_body,
        grid=(E_loc,),
        in_specs=[
            pl.BlockSpec(memory_space=pltpu.SMEM),
            pl.BlockSpec((S, D), lambda e: (0, 0)),
            pl.BlockSpec((D, E), lambda e: (0, 0)),
            pl.BlockSpec((1, H, D), lambda e: (e, 0, 0)),
            pl.BlockSpec((1, H, D), lambda e: (e, 0, 0)),
            pl.BlockSpec((1, D, H), lambda e: (e, 0, 0)),
        ],
        out_specs=pl.BlockSpec((S, D), lambda e: (0, 0)),
        out_shape=jax.ShapeDtypeStruct((S, D), jnp.float32),
        scratch_shapes=[pltpu.VMEM((S, E), jnp.float32)],
    )(off, x, W_gate, up_proj, gate_proj, down_proj)


@jax.jit
def kernel(x, W_gate, up_proj, gate_proj, down_proj):
    E = up_proj.shape[0]
    devs = jax.devices()
    nd = 2 if len(devs) >= 2 and E % 2 == 0 else 1
    E_loc = E // nd
    mesh = Mesh(np.array(devs[:nd]), ("d",))

    def shard_fn(x, wg, up, gp, dp):
        off = (jax.lax.axis_index("d") * E_loc).astype(jnp.int32).reshape((1,))
        out_part = _local_moe(off, x, wg, up, gp, dp)
        return jax.lax.psum(out_part, "d")

    f = _shard_map(
        shard_fn,
        mesh=mesh,
        in_specs=(P(), P(), P("d"), P("d"), P("d")),
        out_specs=P(),
    )
    return f(x, W_gate, up_proj, gate_proj, down_proj)


# speedup vs baseline: 14.8624x; 14.8624x over previous
import functools

import jax
import jax.numpy as jnp
from jax.experimental import pallas as pl
from jax.experimental.pallas import tpu as pltpu

_EPB = 2


def _moe_body(x_ref, wg_ref, up_ref, gate_ref, down_ref, out_ref, w_scr):
    eb = pl.program_id(0)

    @pl.when(eb == 0)
    def _compute_gating():
        xf = x_ref[...].astype(jnp.float32)
        logits = jnp.dot(xf, wg_ref[...], preferred_element_type=jnp.float32)
        m = jnp.max(logits, axis=-1, keepdims=True)
        p = jnp.exp(logits - m)
        g = p / jnp.sum(p, axis=-1, keepdims=True)
        col = jax.lax.broadcasted_iota(jnp.int32, g.shape, 1)
        i1 = jnp.argmax(g, axis=-1)[:, None]
        oh1 = col == i1
        i2 = jnp.argmax(jnp.where(oh1, -1.0, g), axis=-1)[:, None]
        oh2 = col == i2
        w_scr[...] = jnp.where(oh1 | oh2, g, 0.0)

    xb = x_ref[...]
    dn = (((1,), (1,)), ((), ()))
    wcol = jax.lax.broadcasted_iota(jnp.int32, w_scr.shape, 1)
    contrib = None
    for j in range(_EPB):
        up = jax.lax.dot_general(xb, up_ref[j], dn,
                                 preferred_element_type=jnp.float32)
        gate = jax.lax.dot_general(xb, gate_ref[j], dn,
                                   preferred_element_type=jnp.float32)
        hidden = (gate * jax.nn.sigmoid(gate) * up).astype(jnp.bfloat16)
        y = jax.lax.dot_general(hidden, down_ref[j], dn,
                                preferred_element_type=jnp.float32)
        we = jnp.sum(jnp.where(wcol == eb * _EPB + j, w_scr[...], 0.0),
                     axis=1, keepdims=True)
        c = we * y
        contrib = c if contrib is None else contrib + c

    @pl.when(eb == 0)
    def _init():
        out_ref[...] = contrib

    @pl.when(eb > 0)
    def _acc():
        out_ref[...] += contrib


@jax.jit
def kernel(x, W_gate, up_proj, gate_proj, down_proj):
    S, D = x.shape
    E, H, _ = up_proj.shape
    return pl.pallas_call(
        _moe_body,
        grid=(E // _EPB,),
        in_specs=[
            pl.BlockSpec((S, D), lambda e: (0, 0)),
            pl.BlockSpec((D, E), lambda e: (0, 0)),
            pl.BlockSpec((_EPB, H, D), lambda e: (e, 0, 0)),
            pl.BlockSpec((_EPB, H, D), lambda e: (e, 0, 0)),
            pl.BlockSpec((_EPB, D, H), lambda e: (e, 0, 0)),
        ],
        out_specs=pl.BlockSpec((S, D), lambda e: (0, 0)),
        out_shape=jax.ShapeDtypeStruct((S, D), jnp.float32),
        scratch_shapes=[pltpu.VMEM((S, E), jnp.float32)],
    )(x, W_gate, up_proj, gate_proj, down_proj)


# speedup vs baseline: 15.7246x; 1.0580x over previous
import functools

import jax
import jax.numpy as jnp
from jax.experimental import pallas as pl
from jax.experimental.pallas import tpu as pltpu


def _moe_body(x_ref, wg_ref, up_hbm, gate_hbm, down_hbm, out_ref,
              ubuf, gbuf, dbuf, sems):
    E = wg_ref.shape[1]

    xf = x_ref[...].astype(jnp.float32)
    logits = jnp.dot(xf, wg_ref[...], preferred_element_type=jnp.float32)
    m = jnp.max(logits, axis=-1, keepdims=True)
    p = jnp.exp(logits - m)
    g = p / jnp.sum(p, axis=-1, keepdims=True)
    col = jax.lax.broadcasted_iota(jnp.int32, g.shape, 1)
    i1 = jnp.argmax(g, axis=-1)[:, None]
    oh1 = col == i1
    i2 = jnp.argmax(jnp.where(oh1, -1.0, g), axis=-1)[:, None]
    oh2 = col == i2
    w = jnp.where(oh1 | oh2, g, 0.0)

    def copies(e):
        slot = e % 2
        return (
            pltpu.make_async_copy(up_hbm.at[e], ubuf.at[slot], sems.at[0, slot]),
            pltpu.make_async_copy(gate_hbm.at[e], gbuf.at[slot], sems.at[1, slot]),
            pltpu.make_async_copy(down_hbm.at[e], dbuf.at[slot], sems.at[2, slot]),
        )

    for c in copies(0):
        c.start()
    for c in copies(1):
        c.start()

    xb = x_ref[...]
    dn = (((1,), (1,)), ((), ()))
    acc = None
    for e in range(E):
        slot = e % 2
        cu, cg, cd = copies(e)
        cu.wait()
        up = jax.lax.dot_general(xb, ubuf[slot], dn,
                                 preferred_element_type=jnp.float32)
        cg.wait()
        gate = jax.lax.dot_general(xb, gbuf[slot], dn,
                                   preferred_element_type=jnp.float32)
        hidden = (gate * jax.nn.sigmoid(gate) * up).astype(jnp.bfloat16)
        cd.wait()
        y = jax.lax.dot_general(hidden, dbuf[slot], dn,
                                preferred_element_type=jnp.float32)
        we = jnp.sum(jnp.where(col == e, w, 0.0), axis=1, keepdims=True)
        c = we * y
        acc = c if acc is None else acc + c
        if e + 2 < E:
            for cp in copies(e + 2):
                cp.start()

    out_ref[...] = acc


@jax.jit
def kernel(x, W_gate, up_proj, gate_proj, down_proj):
    S, D = x.shape
    E, H, _ = up_proj.shape
    return pl.pallas_call(
        _moe_body,
        in_specs=[
            pl.BlockSpec(memory_space=pltpu.VMEM),
            pl.BlockSpec(memory_space=pltpu.VMEM),
            pl.BlockSpec(memory_space=pl.ANY),
            pl.BlockSpec(memory_space=pl.ANY),
            pl.BlockSpec(memory_space=pl.ANY),
        ],
        out_specs=pl.BlockSpec(memory_space=pltpu.VMEM),
        out_shape=jax.ShapeDtypeStruct((S, D), jnp.float32),
        scratch_shapes=[
            pltpu.VMEM((2, H, D), jnp.bfloat16),
            pltpu.VMEM((2, H, D), jnp.bfloat16),
            pltpu.VMEM((2, D, H), jnp.bfloat16),
            pltpu.SemaphoreType.DMA((3, 2)),
        ],
    )(x, W_gate, up_proj, gate_proj, down_proj)


# speedup vs baseline: 16.0821x; 1.0227x over previous
import functools

import jax
import jax.numpy as jnp
from jax.experimental import pallas as pl
from jax.experimental.pallas import tpu as pltpu


def _moe_body(x_ref, wg_ref, up_hbm, gate_hbm, down_hbm, out_ref,
              ubuf, gbuf, dbuf, sems):
    E = wg_ref.shape[1]

    xf = x_ref[...].astype(jnp.float32)
    logits = jnp.dot(xf, wg_ref[...], preferred_element_type=jnp.float32)
    m = jnp.max(logits, axis=-1, keepdims=True)
    p = jnp.exp(logits - m)
    g = p / jnp.sum(p, axis=-1, keepdims=True)
    col = jax.lax.broadcasted_iota(jnp.int32, g.shape, 1)
    i1 = jnp.argmax(g, axis=-1)[:, None]
    oh1 = col == i1
    i2 = jnp.argmax(jnp.where(oh1, -1.0, g), axis=-1)[:, None]
    oh2 = col == i2
    w = jnp.where(oh1 | oh2, g, 0.0)

    def copies(e):
        slot = e % 2
        return (
            pltpu.make_async_copy(up_hbm.at[e], ubuf.at[slot], sems.at[0, slot]),
            pltpu.make_async_copy(gate_hbm.at[e], gbuf.at[slot], sems.at[1, slot]),
            pltpu.make_async_copy(down_hbm.at[e], dbuf.at[slot], sems.at[2, slot]),
        )

    for c in copies(0):
        c.start()
    for c in copies(1):
        c.start()

    xb = x_ref[...]
    dn = (((1,), (1,)), ((), ()))
    acc = None
    for e in range(E):
        slot = e % 2
        cu, cg, cd = copies(e)
        cu.wait()
        cg.wait()
        cd.wait()
        H = ubuf.shape[1]
        HC = 512
        y = None
        for cs in range(0, H, HC):
            up_c = jax.lax.dot_general(xb, ubuf[slot, cs:cs + HC, :], dn,
                                       preferred_element_type=jnp.float32)
            gate_c = jax.lax.dot_general(xb, gbuf[slot, cs:cs + HC, :], dn,
                                         preferred_element_type=jnp.float32)
            hidden_c = (gate_c * jax.nn.sigmoid(gate_c) * up_c).astype(jnp.bfloat16)
            y_c = jax.lax.dot_general(hidden_c, dbuf[slot, :, cs:cs + HC], dn,
                                      preferred_element_type=jnp.float32)
            y = y_c if y is None else y + y_c
        we = jnp.sum(jnp.where(col == e, w, 0.0), axis=1, keepdims=True)
        c = we * y
        acc = c if acc is None else acc + c
        if e + 2 < E:
            for cp in copies(e + 2):
                cp.start()

    out_ref[...] = acc


@jax.jit
def kernel(x, W_gate, up_proj, gate_proj, down_proj):
    S, D = x.shape
    E, H, _ = up_proj.shape
    return pl.pallas_call(
        _moe_body,
        in_specs=[
            pl.BlockSpec(memory_space=pltpu.VMEM),
            pl.BlockSpec(memory_space=pltpu.VMEM),
            pl.BlockSpec(memory_space=pl.ANY),
            pl.BlockSpec(memory_space=pl.ANY),
            pl.BlockSpec(memory_space=pl.ANY),
        ],
        out_specs=pl.BlockSpec(memory_space=pltpu.VMEM),
        out_shape=jax.ShapeDtypeStruct((S, D), jnp.float32),
        scratch_shapes=[
            pltpu.VMEM((2, H, D), jnp.bfloat16),
            pltpu.VMEM((2, H, D), jnp.bfloat16),
            pltpu.VMEM((2, D, H), jnp.bfloat16),
            pltpu.SemaphoreType.DMA((3, 2)),
        ],
    )(x, W_gate, up_proj, gate_proj, down_proj)


# speedup vs baseline: 16.1259x; 1.0027x over previous
import functools

import jax
import jax.numpy as jnp
from jax.experimental import pallas as pl
from jax.experimental.pallas import tpu as pltpu


def _moe_body(x_ref, wg_ref, up_hbm, gate_hbm, down_hbm, out_ref,
              ubuf, gbuf, dbuf, sems):
    E = wg_ref.shape[1]

    xf = x_ref[...].astype(jnp.float32)
    logits = jnp.dot(xf, wg_ref[...], preferred_element_type=jnp.float32)
    m = jnp.max(logits, axis=-1, keepdims=True)
    p = jnp.exp(logits - m)
    g = p / jnp.sum(p, axis=-1, keepdims=True)
    col = jax.lax.broadcasted_iota(jnp.int32, g.shape, 1)
    i1 = jnp.argmax(g, axis=-1)[:, None]
    oh1 = col == i1
    i2 = jnp.argmax(jnp.where(oh1, -1.0, g), axis=-1)[:, None]
    oh2 = col == i2
    w = jnp.where(oh1 | oh2, g, 0.0)

    def copies(e):
        slot = e % 3
        return (
            pltpu.make_async_copy(up_hbm.at[e], ubuf.at[slot], sems.at[0, slot]),
            pltpu.make_async_copy(gate_hbm.at[e], gbuf.at[slot], sems.at[1, slot]),
            pltpu.make_async_copy(down_hbm.at[e], dbuf.at[slot], sems.at[2, slot]),
        )

    for c in copies(0):
        c.start()
    for c in copies(1):
        c.start()
    for c in copies(2):
        c.start()

    xb = x_ref[...]
    dn = (((1,), (1,)), ((), ()))
    acc = None
    for e in range(E):
        slot = e % 3
        cu, cg, cd = copies(e)
        H = ubuf.shape[1]
        HC = 512
        cu.wait()
        cg.wait()
        cd.wait()
        y = None
        for cs in range(0, H, HC):
            up_c = jax.lax.dot_general(xb, ubuf[slot, cs:cs + HC, :], dn,
                                       preferred_element_type=jnp.float32)
            gate_c = jax.lax.dot_general(xb, gbuf[slot, cs:cs + HC, :], dn,
                                         preferred_element_type=jnp.float32)
            hidden_c = (gate_c * jax.nn.sigmoid(gate_c) * up_c).astype(jnp.bfloat16)
            y_c = jax.lax.dot_general(hidden_c, dbuf[slot, :, cs:cs + HC], dn,
                                      preferred_element_type=jnp.float32)
            y = y_c if y is None else y + y_c
        we = jnp.sum(jnp.where(col == e, w, 0.0), axis=1, keepdims=True)
        c = we * y
        acc = c if acc is None else acc + c
        if e + 3 < E:
            for cp in copies(e + 3):
                cp.start()

    out_ref[...] = acc


@jax.jit
def kernel(x, W_gate, up_proj, gate_proj, down_proj):
    S, D = x.shape
    E, H, _ = up_proj.shape
    return pl.pallas_call(
        _moe_body,
        in_specs=[
            pl.BlockSpec(memory_space=pltpu.VMEM),
            pl.BlockSpec(memory_space=pltpu.VMEM),
            pl.BlockSpec(memory_space=pl.ANY),
            pl.BlockSpec(memory_space=pl.ANY),
            pl.BlockSpec(memory_space=pl.ANY),
        ],
        out_specs=pl.BlockSpec(memory_space=pltpu.VMEM),
        out_shape=jax.ShapeDtypeStruct((S, D), jnp.float32),
        scratch_shapes=[
            pltpu.VMEM((3, H, D), jnp.bfloat16),
            pltpu.VMEM((3, H, D), jnp.bfloat16),
            pltpu.VMEM((3, D, H), jnp.bfloat16),
            pltpu.SemaphoreType.DMA((3, 3)),
        ],
    )(x, W_gate, up_proj, gate_proj, down_proj)


# speedup vs baseline: 17.3788x; 1.0777x over previous
import functools

import jax
import jax.numpy as jnp
from jax.experimental import pallas as pl
from jax.experimental.pallas import tpu as pltpu


def _moe_body(x_ref, wg_ref, up_hbm, gate_hbm, down_hbm, out_ref,
              ubuf, gbuf, dbuf, sems):
    E = wg_ref.shape[1]

    xf = x_ref[...].astype(jnp.float32)
    logits = jnp.dot(xf, wg_ref[...], preferred_element_type=jnp.float32)
    m = jnp.max(logits, axis=-1, keepdims=True)
    p = jnp.exp(logits - m)
    g = p / jnp.sum(p, axis=-1, keepdims=True)
    col = jax.lax.broadcasted_iota(jnp.int32, g.shape, 1)
    i1 = jnp.argmax(g, axis=-1)[:, None]
    oh1 = col == i1
    i2 = jnp.argmax(jnp.where(oh1, -1.0, g), axis=-1)[:, None]
    oh2 = col == i2
    w = jnp.where(oh1 | oh2, g, 0.0)

    def copies(e):
        slot = e % 3
        return (
            pltpu.make_async_copy(up_hbm.at[e], ubuf.at[slot], sems.at[0, slot]),
            pltpu.make_async_copy(gate_hbm.at[e], gbuf.at[slot], sems.at[1, slot]),
            pltpu.make_async_copy(down_hbm.at[e], dbuf.at[slot], sems.at[2, slot]),
        )

    for c in copies(0):
        c.start()
    for c in copies(1):
        c.start()
    for c in copies(2):
        c.start()

    xb = x_ref[...]
    dn = (((1,), (1,)), ((), ()))
    H = ubuf.shape[1]
    HC = 512
    NC = H // HC
    wes = [jnp.sum(jnp.where(col == e, w, 0.0), axis=1, keepdims=True)
           for e in range(E)]
    acc = None
    pend = None
    for k in range(E * NC):
        e, c = divmod(k, NC)
        slot = e % 3
        cs = c * HC
        if c == 0:
            cu, cg, cd = copies(e)
            cu.wait()
            cg.wait()
            cd.wait()
        gate_c = jax.lax.dot_general(xb, gbuf[slot, cs:cs + HC, :], dn,
                                     preferred_element_type=jnp.float32)
        silu_c = gate_c * jax.nn.sigmoid(gate_c)
        up_c = jax.lax.dot_general(xb, ubuf[slot, cs:cs + HC, :], dn,
                                   preferred_element_type=jnp.float32)
        hidden_c = (silu_c * up_c).astype(jnp.bfloat16)
        if pend is not None:
            ph, pslot, pcs, pe = pend
            y_c = jax.lax.dot_general(ph, dbuf[pslot, :, pcs:pcs + HC], dn,
                                      preferred_element_type=jnp.float32)
            contrib = wes[pe] * y_c
            acc = contrib if acc is None else acc + contrib
        pend = (hidden_c, slot, cs, e)
        if c == NC - 1 and e + 3 < E:
            for cp in copies(e + 3):
                cp.start()
    ph, pslot, pcs, pe = pend
    y_c = jax.lax.dot_general(ph, dbuf[pslot, :, pcs:pcs + HC], dn,
                              preferred_element_type=jnp.float32)
    acc = acc + wes[pe] * y_c

    out_ref[...] = acc


@jax.jit
def kernel(x, W_gate, up_proj, gate_proj, down_proj):
    S, D = x.shape
    E, H, _ = up_proj.shape
    return pl.pallas_call(
        _moe_body,
        in_specs=[
            pl.BlockSpec(memory_space=pltpu.VMEM),
            pl.BlockSpec(memory_space=pltpu.VMEM),
            pl.BlockSpec(memory_space=pl.ANY),
            pl.BlockSpec(memory_space=pl.ANY),
            pl.BlockSpec(memory_space=pl.ANY),
        ],
        out_specs=pl.BlockSpec(memory_space=pltpu.VMEM),
        out_shape=jax.ShapeDtypeStruct((S, D), jnp.float32),
        scratch_shapes=[
            pltpu.VMEM((3, H, D), jnp.bfloat16),
            pltpu.VMEM((3, H, D), jnp.bfloat16),
            pltpu.VMEM((3, D, H), jnp.bfloat16),
            pltpu.SemaphoreType.DMA((3, 3)),
        ],
    )(x, W_gate, up_proj, gate_proj, down_proj)


# speedup vs baseline: 17.4970x; 1.0068x over previous
import functools

import jax
import jax.numpy as jnp
from jax.experimental import pallas as pl
from jax.experimental.pallas import tpu as pltpu


def _moe_body(x_ref, wg_ref, up_hbm, gate_hbm, down_hbm, out_ref,
              ubuf, gbuf, dbuf, sems):
    E = wg_ref.shape[1]

    xf = x_ref[...].astype(jnp.float32)
    logits = jnp.dot(xf, wg_ref[...], preferred_element_type=jnp.float32)
    m = jnp.max(logits, axis=-1, keepdims=True)
    p = jnp.exp(logits - m)
    g = p / jnp.sum(p, axis=-1, keepdims=True)
    col = jax.lax.broadcasted_iota(jnp.int32, g.shape, 1)
    i1 = jnp.argmax(g, axis=-1)[:, None]
    oh1 = col == i1
    i2 = jnp.argmax(jnp.where(oh1, -1.0, g), axis=-1)[:, None]
    oh2 = col == i2
    w = jnp.where(oh1 | oh2, g, 0.0)

    def copies(e):
        slot = e % 3
        return (
            pltpu.make_async_copy(up_hbm.at[e], ubuf.at[slot], sems.at[0, slot]),
            pltpu.make_async_copy(gate_hbm.at[e], gbuf.at[slot], sems.at[1, slot]),
            pltpu.make_async_copy(down_hbm.at[e], dbuf.at[slot], sems.at[2, slot]),
        )

    for c in copies(0):
        c.start()
    for c in copies(1):
        c.start()
    for c in copies(2):
        c.start()

    xb = x_ref[...]
    dn = (((1,), (1,)), ((), ()))
    H = ubuf.shape[1]
    HC = 512
    NC = H // HC
    wes = [jnp.sum(jnp.where(col == e, w, 0.0), axis=1, keepdims=True)
           for e in range(E)]
    acc = None
    pend = None
    for k in range(E * NC):
        e, c = divmod(k, NC)
        slot = e % 3
        cs = c * HC
        if c == 0:
            cu, cg, cd = copies(e)
            cu.wait()
            cg.wait()
            cd.wait()
        gate_c = jax.lax.dot_general(xb, gbuf[slot, cs:cs + HC, :], dn,
                                     preferred_element_type=jnp.float32)
        silu_c = gate_c * jax.nn.sigmoid(gate_c)
        up_c = jax.lax.dot_general(xb, ubuf[slot, cs:cs + HC, :], dn,
                                   preferred_element_type=jnp.float32)
        hidden_c = (silu_c * up_c).astype(jnp.bfloat16)
        if pend is not None:
            ph, pslot, pcs, pe = pend
            y_c = jax.lax.dot_general(ph, dbuf[pslot, :, pcs:pcs + HC], dn,
                                      preferred_element_type=jnp.float32)
            contrib = wes[pe] * y_c
            acc = contrib if acc is None else acc + contrib
        pend = (hidden_c, slot, cs, e)
        if c == 0 and e >= 1 and e + 2 < E:
            for cp in copies(e + 2):
                cp.start()
    ph, pslot, pcs, pe = pend
    y_c = jax.lax.dot_general(ph, dbuf[pslot, :, pcs:pcs + HC], dn,
                              preferred_element_type=jnp.float32)
    acc = acc + wes[pe] * y_c

    out_ref[...] = acc


@jax.jit
def kernel(x, W_gate, up_proj, gate_proj, down_proj):
    S, D = x.shape
    E, H, _ = up_proj.shape
    return pl.pallas_call(
        _moe_body,
        in_specs=[
            pl.BlockSpec(memory_space=pltpu.VMEM),
            pl.BlockSpec(memory_space=pltpu.VMEM),
            pl.BlockSpec(memory_space=pl.ANY),
            pl.BlockSpec(memory_space=pl.ANY),
            pl.BlockSpec(memory_space=pl.ANY),
        ],
        out_specs=pl.BlockSpec(memory_space=pltpu.VMEM),
        out_shape=jax.ShapeDtypeStruct((S, D), jnp.float32),
        scratch_shapes=[
            pltpu.VMEM((3, H, D), jnp.bfloat16),
            pltpu.VMEM((3, H, D), jnp.bfloat16),
            pltpu.VMEM((3, D, H), jnp.bfloat16),
            pltpu.SemaphoreType.DMA((3, 3)),
        ],
    )(x, W_gate, up_proj, gate_proj, down_proj)
